# Initial kernel scaffold; baseline (speedup 1.0000x reference)
import jax
import jax.numpy as jnp
from jax.experimental import pallas as pl


def kernel(x, edge_index, edge_attr, batch, W1, b1, W2, b2, Wl, bl):
    raise NotImplementedError("write your pallas kernel here")


# speedup vs baseline: 7.9532x; 7.9532x over previous
import functools

import jax
import jax.numpy as jnp
from jax import lax
from jax.experimental import pallas as pl
from jax.experimental.pallas import tpu as pltpu
from jax.experimental.pallas import tpu_sc as plsc

NC = 2
NS = 16
NW = NC * NS
CHUNK = 128
NP = 10240
EP = 327680
EPW = EP // NW
NPT = NP // NS
BM = 2048
NB = NP // BM
NG = 16
D = 128

_PREC = lax.Precision.HIGHEST


def _mesh():
    return plsc.VectorSubcoreMesh(core_axis_name="c", subcore_axis_name="s")


def _sc_count(dst, zeros128):
    @functools.partial(
        pl.kernel,
        out_type=jax.ShapeDtypeStruct((NC, NP, D), jnp.float32),
        mesh=_mesh(),
        scratch_types=[
            pltpu.VMEM((CHUNK,), jnp.int32),
            pltpu.VMEM((CHUNK, D), jnp.float32),
            pltpu.VMEM_SHARED((NP, D), jnp.float32),
        ],
    )
    def body(dst_hbm, z_hbm, out_hbm, idx_v, ones_v, acc_sh):
        c = lax.axis_index("c")
        s = lax.axis_index("s")
        wid = s * NC + c

        def fill(r, carry):
            def fill16(j, carry2):
                ones_v[r, pl.ds(j * 16, 16)] = jnp.full((16,), 1.0,
                                                        jnp.float32)
                return carry2
            return lax.fori_loop(0, D // 16, fill16, carry)

        lax.fori_loop(0, CHUNK, fill, 0)
        pltpu.sync_copy(z_hbm.at[pl.ds(s * NPT, NPT)],
                        acc_sh.at[pl.ds(s * NPT, NPT)])
        plsc.subcore_barrier()
        base = wid * EPW

        def step(k, carry):
            pltpu.sync_copy(dst_hbm.at[pl.ds(base + k * CHUNK, CHUNK)], idx_v)
            pltpu.sync_copy(ones_v, acc_sh.at[idx_v], add=True)
            return carry

        lax.fori_loop(0, EPW // CHUNK, step, 0)
        plsc.subcore_barrier()
        pltpu.sync_copy(acc_sh.at[pl.ds(s * NPT, NPT)],
                        out_hbm.at[c, pl.ds(s * NPT, NPT)])

    return body(dst, zeros128)


def _sc_agg(src, dst, g, zeros128):
    @functools.partial(
        pl.kernel,
        out_type=jax.ShapeDtypeStruct((NC, NP, D), jnp.float32),
        mesh=_mesh(),
        scratch_types=[
            pltpu.VMEM((CHUNK,), jnp.int32),
            pltpu.VMEM((CHUNK,), jnp.int32),
            pltpu.VMEM((CHUNK, D), jnp.float32),
            pltpu.VMEM_SHARED((NP, D), jnp.float32),
            pltpu.SemaphoreType.DMA,
        ],
    )
    def body(src_hbm, dst_hbm, g_hbm, z_hbm, out_hbm,
             sidx_v, didx_v, rows_v, acc_sh, sem):
        c = lax.axis_index("c")
        s = lax.axis_index("s")
        wid = s * NC + c
        r0 = s * NPT

        @pl.when(c == 0)
        def _():
            pltpu.sync_copy(g_hbm.at[pl.ds(r0, NPT)], acc_sh.at[pl.ds(r0, NPT)])

        @pl.when(c != 0)
        def _():
            pltpu.sync_copy(z_hbm.at[pl.ds(r0, NPT)], acc_sh.at[pl.ds(r0, NPT)])

        plsc.subcore_barrier()
        base = wid * EPW

        def step(k, carry):
            pltpu.sync_copy(src_hbm.at[pl.ds(base + k * CHUNK, CHUNK)], sidx_v)
            pltpu.sync_copy(dst_hbm.at[pl.ds(base + k * CHUNK, CHUNK)], didx_v)
            pltpu.async_copy(g_hbm.at[sidx_v], rows_v, sem).wait()
            pltpu.sync_copy(rows_v, acc_sh.at[didx_v], add=True)
            return carry

        lax.fori_loop(0, EPW // CHUNK, step, 0)
        plsc.subcore_barrier()
        pltpu.sync_copy(acc_sh.at[pl.ds(r0, NPT)],
                        out_hbm.at[c, pl.ds(r0, NPT)])

    return body(src, dst, g, zeros128)


def _tc_g1(x_pad, W1, cnt):
    def body(x_ref, w_ref, cnt_ref, g_ref, dinv_ref):
        cntv = cnt_ref[0, :, 0] + cnt_ref[1, :, 0] + 1.0
        dinv = lax.rsqrt(cntv)
        h = jnp.dot(x_ref[...], w_ref[...], precision=_PREC,
                    preferred_element_type=jnp.float32)
        g_ref[...] = h * dinv[:, None]
        dinv_ref[0, 0, :] = dinv

    return pl.pallas_call(
        body,
        grid=(NB,),
        in_specs=[
            pl.BlockSpec((BM, D), lambda i: (i, 0)),
            pl.BlockSpec((D, D), lambda i: (0, 0)),
            pl.BlockSpec((NC, BM, D), lambda i: (0, i, 0)),
        ],
        out_specs=[
            pl.BlockSpec((BM, D), lambda i: (i, 0)),
            pl.BlockSpec((1, 1, BM), lambda i: (i, 0, 0)),
        ],
        out_shape=[
            jax.ShapeDtypeStruct((NP, D), jnp.float32),
            jax.ShapeDtypeStruct((NB, 1, BM), jnp.float32),
        ],
    )(x_pad, W1, cnt)


def _tc_combine(agg, dinv3d, b1r, W2):
    def body(a_ref, dinv_ref, b_ref, w_ref, g_ref):
        dinv = dinv_ref[0, 0, :]
        p = a_ref[0] + a_ref[1]
        h = jnp.maximum(p * dinv[:, None] + b_ref[...], 0.0)
        g_ref[...] = jnp.dot(h, w_ref[...], precision=_PREC,
                             preferred_element_type=jnp.float32) * dinv[:, None]

    return pl.pallas_call(
        body,
        grid=(NB,),
        in_specs=[
            pl.BlockSpec((NC, BM, D), lambda i: (0, i, 0)),
            pl.BlockSpec((1, 1, BM), lambda i: (i, 0, 0)),
            pl.BlockSpec((1, D), lambda i: (0, 0)),
            pl.BlockSpec((D, D), lambda i: (0, 0)),
        ],
        out_specs=pl.BlockSpec((BM, D), lambda i: (i, 0)),
        out_shape=jax.ShapeDtypeStruct((NP, D), jnp.float32),
    )(agg, dinv3d, b1r, W2)


def _tc_final(agg2, dinv3d, b2r, batch3d, Wl, blr):
    def body(a_ref, dinv_ref, b_ref, bat_ref, wl_ref, bl_ref, out_ref,
             sums, cnts):
        i = pl.program_id(0)

        @pl.when(i == 0)
        def _():
            sums[...] = jnp.zeros_like(sums)
            cnts[...] = jnp.zeros_like(cnts)

        dinv = dinv_ref[0, 0, :]
        h2 = (a_ref[0] + a_ref[1]) * dinv[:, None] + b_ref[...]
        bat = bat_ref[0, 0, :]
        gids = lax.broadcasted_iota(jnp.int32, (1, NG), 1)
        mask = (bat[:, None] == gids).astype(jnp.float32)
        sums[...] += lax.dot_general(mask, h2, (((0,), (0,)), ((), ())),
                                     precision=_PREC,
                                     preferred_element_type=jnp.float32)
        cnts[...] += lax.dot_general(mask, jnp.ones_like(h2),
                                     (((0,), (0,)), ((), ())),
                                     precision=_PREC,
                                     preferred_element_type=jnp.float32)

        @pl.when(i == NB - 1)
        def _():
            pooled = sums[...] / jnp.maximum(cnts[...], 1.0)
            out_ref[...] = jnp.dot(pooled, wl_ref[...], precision=_PREC,
                                   preferred_element_type=jnp.float32) + bl_ref[...]

    return pl.pallas_call(
        body,
        grid=(NB,),
        in_specs=[
            pl.BlockSpec((NC, BM, D), lambda i: (0, i, 0)),
            pl.BlockSpec((1, 1, BM), lambda i: (i, 0, 0)),
            pl.BlockSpec((1, D), lambda i: (0, 0)),
            pl.BlockSpec((1, 1, BM), lambda i: (i, 0, 0)),
            pl.BlockSpec((D, D), lambda i: (0, 0)),
            pl.BlockSpec((1, D), lambda i: (0, 0)),
        ],
        out_specs=pl.BlockSpec((NG, D), lambda i: (0, 0)),
        out_shape=jax.ShapeDtypeStruct((NG, D), jnp.float32),
        scratch_shapes=[
            pltpu.VMEM((NG, D), jnp.float32),
            pltpu.VMEM((NG, D), jnp.float32),
        ],
    )(agg2, dinv3d, b2r, batch3d, Wl, blr)


@jax.jit
def _run(x, edge_index, batch, W1, b1, W2, b2, Wl, bl):
    n = x.shape[0]
    e = edge_index.shape[1]

    pad_idx = jnp.full((EP - e,), n, dtype=jnp.int32)
    src = jnp.concatenate([edge_index[0].astype(jnp.int32), pad_idx])
    dst = jnp.concatenate([edge_index[1].astype(jnp.int32), pad_idx])
    x_pad = jnp.concatenate(
        [x, jnp.zeros((NP - n, x.shape[1]), dtype=x.dtype)])
    batch_pad = jnp.concatenate(
        [batch.astype(jnp.int32), jnp.full((NP - n,), NG, dtype=jnp.int32)])
    batch3d = batch_pad.reshape(NB, 1, BM)

    zeros128 = jnp.zeros((NP, D), dtype=jnp.float32)
    b1r = b1.reshape(1, D)
    b2r = b2.reshape(1, D)
    blr = bl.reshape(1, D)

    cnt = _sc_count(dst, zeros128)
    g1, dinv3d = _tc_g1(x_pad, W1, cnt)
    agg1 = _sc_agg(src, dst, g1, zeros128)
    g2 = _tc_combine(agg1, dinv3d, b1r, W2)
    agg2 = _sc_agg(src, dst, g2, zeros128)
    return _tc_final(agg2, dinv3d, b2r, batch3d, Wl, blr)


def kernel(x, edge_index, edge_attr, batch, W1, b1, W2, b2, Wl, bl):
    del edge_attr
    return _run(x, edge_index, batch, W1, b1, W2, b2, Wl, bl)


# speedup vs baseline: 8.9785x; 1.1289x over previous
import functools

import jax
import jax.numpy as jnp
from jax import lax
from jax.experimental import pallas as pl
from jax.experimental.pallas import tpu as pltpu
from jax.experimental.pallas import tpu_sc as plsc

NC = 2
NS = 16
NW = NC * NS
CHUNK = 128
NP = 10240
EP = 327680
EPW = EP // NW
NPT = NP // NS
BM = 2048
NB = NP // BM
NG = 16
D = 128

_PREC = lax.Precision.HIGHEST


def _mesh():
    return plsc.VectorSubcoreMesh(core_axis_name="c", subcore_axis_name="s")


def _sc_count(dst, zeros128):
    @functools.partial(
        pl.kernel,
        out_type=jax.ShapeDtypeStruct((NC, NP, D), jnp.float32),
        mesh=_mesh(),
        scratch_types=[
            pltpu.VMEM((CHUNK,), jnp.int32),
            pltpu.VMEM((CHUNK, D), jnp.float32),
            pltpu.VMEM_SHARED((NP, D), jnp.float32),
        ],
    )
    def body(dst_hbm, z_hbm, out_hbm, idx_v, ones_v, acc_sh):
        c = lax.axis_index("c")
        s = lax.axis_index("s")
        wid = s * NC + c

        def fill(r, carry):
            def fill16(j, carry2):
                ones_v[r, pl.ds(j * 16, 16)] = jnp.full((16,), 1.0,
                                                        jnp.float32)
                return carry2
            return lax.fori_loop(0, D // 16, fill16, carry)

        lax.fori_loop(0, CHUNK, fill, 0)
        pltpu.sync_copy(z_hbm.at[pl.ds(s * NPT, NPT)],
                        acc_sh.at[pl.ds(s * NPT, NPT)])
        plsc.subcore_barrier()
        base = wid * EPW

        def step(k, carry):
            pltpu.sync_copy(dst_hbm.at[pl.ds(base + k * CHUNK, CHUNK)], idx_v)
            pltpu.sync_copy(ones_v, acc_sh.at[idx_v], add=True)
            return carry

        lax.fori_loop(0, EPW // CHUNK, step, 0)
        plsc.subcore_barrier()
        pltpu.sync_copy(acc_sh.at[pl.ds(s * NPT, NPT)],
                        out_hbm.at[c, pl.ds(s * NPT, NPT)])

    return body(dst, zeros128)


def _sc_agg(src, dst, g, zeros128):
    nk = EPW // CHUNK
    GK = 8
    ngrp = nk // GK

    @functools.partial(
        pl.kernel,
        out_type=jax.ShapeDtypeStruct((NC, NP, D), jnp.float32),
        mesh=_mesh(),
        scratch_types=[
            pltpu.VMEM((2, GK, CHUNK), jnp.int32),
            pltpu.VMEM((2, GK, CHUNK), jnp.int32),
            pltpu.VMEM((CHUNK, D), jnp.float32),
            pltpu.VMEM((CHUNK, D), jnp.float32),
            pltpu.SemaphoreType.DMA,
            pltpu.SemaphoreType.DMA,
            pltpu.SemaphoreType.DMA,
            pltpu.SemaphoreType.DMA,
            pltpu.SemaphoreType.DMA,
            pltpu.VMEM_SHARED((NP, D), jnp.float32),
        ],
    )
    def body(src_hbm, dst_hbm, g_hbm, z_hbm, out_hbm,
             sidx_v, didx_v, rows_a, rows_b, gsa, gsb, ssa, ssb, isem,
             acc_sh):
        c = lax.axis_index("c")
        s = lax.axis_index("s")
        wid = s * NC + c
        r0 = s * NPT
        base = wid * nk
        rows = (rows_a, rows_b)
        gsem = (gsa, gsb)
        ssem = (ssa, ssb)

        def load_group(gi):
            gb = gi % 2
            pltpu.async_copy(src_hbm.at[pl.ds(base + gi * GK, GK)],
                             sidx_v.at[gb], isem)
            pltpu.async_copy(dst_hbm.at[pl.ds(base + gi * GK, GK)],
                             didx_v.at[gb], isem)

        def wait_group(gi):
            gb = gi % 2
            pltpu.make_async_copy(src_hbm.at[pl.ds(base + gi * GK, GK)],
                                  sidx_v.at[gb], isem).wait()
            pltpu.make_async_copy(dst_hbm.at[pl.ds(base + gi * GK, GK)],
                                  didx_v.at[gb], isem).wait()

        load_group(0)

        @pl.when(c == 0)
        def _():
            pltpu.sync_copy(g_hbm.at[pl.ds(r0, NPT)], acc_sh.at[pl.ds(r0, NPT)])

        @pl.when(c != 0)
        def _():
            pltpu.sync_copy(z_hbm.at[pl.ds(r0, NPT)], acc_sh.at[pl.ds(r0, NPT)])

        wait_group(0)
        plsc.subcore_barrier()

        pltpu.async_copy(g_hbm.at[sidx_v.at[0, 0]], rows_a, gsa)
        for k in range(nk):
            b = k & 1
            gi, r = divmod(k, GK)
            if k >= 1 and k + 1 < nk:
                pltpu.make_async_copy(
                    rows[1 - b],
                    acc_sh.at[didx_v.at[(k - 1) // GK % 2, (k - 1) % GK]],
                    ssem[1 - b]).wait()
            if r == 1 and gi + 1 < ngrp:
                load_group(gi + 1)
            if k + 1 < nk:
                ngi, nr = divmod(k + 1, GK)
                if nr == 0:
                    wait_group(ngi)
                pltpu.async_copy(g_hbm.at[sidx_v.at[ngi % 2, nr]],
                                 rows[1 - b], gsem[1 - b])
            pltpu.make_async_copy(g_hbm.at[sidx_v.at[gi % 2, r]],
                                  rows[b], gsem[b]).wait()
            pltpu.async_copy(rows[b], acc_sh.at[didx_v.at[gi % 2, r]],
                             ssem[b], add=True)
        for k in (nk - 2, nk - 1):
            b = k & 1
            gi, r = divmod(k, GK)
            pltpu.make_async_copy(rows[b], acc_sh.at[didx_v.at[gi % 2, r]],
                                  ssem[b]).wait()
        plsc.subcore_barrier()
        pltpu.sync_copy(acc_sh.at[pl.ds(r0, NPT)],
                        out_hbm.at[c, pl.ds(r0, NPT)])

    return body(src.reshape(EP // CHUNK, CHUNK),
                dst.reshape(EP // CHUNK, CHUNK), g, zeros128)


def _tc_g1(x_pad, W1, cnt):
    def body(x_ref, w_ref, cnt_ref, g_ref, dinv_ref):
        cntv = cnt_ref[0, :, 0] + cnt_ref[1, :, 0] + 1.0
        dinv = lax.rsqrt(cntv)
        h = jnp.dot(x_ref[...], w_ref[...], precision=_PREC,
                    preferred_element_type=jnp.float32)
        g_ref[...] = h * dinv[:, None]
        dinv_ref[0, 0, :] = dinv

    return pl.pallas_call(
        body,
        grid=(NB,),
        in_specs=[
            pl.BlockSpec((BM, D), lambda i: (i, 0)),
            pl.BlockSpec((D, D), lambda i: (0, 0)),
            pl.BlockSpec((NC, BM, D), lambda i: (0, i, 0)),
        ],
        out_specs=[
            pl.BlockSpec((BM, D), lambda i: (i, 0)),
            pl.BlockSpec((1, 1, BM), lambda i: (i, 0, 0)),
        ],
        out_shape=[
            jax.ShapeDtypeStruct((NP, D), jnp.float32),
            jax.ShapeDtypeStruct((NB, 1, BM), jnp.float32),
        ],
    )(x_pad, W1, cnt)


def _tc_combine(agg, dinv3d, b1r, W2):
    def body(a_ref, dinv_ref, b_ref, w_ref, g_ref):
        dinv = dinv_ref[0, 0, :]
        p = a_ref[0] + a_ref[1]
        h = jnp.maximum(p * dinv[:, None] + b_ref[...], 0.0)
        g_ref[...] = jnp.dot(h, w_ref[...], precision=_PREC,
                             preferred_element_type=jnp.float32) * dinv[:, None]

    return pl.pallas_call(
        body,
        grid=(NB,),
        in_specs=[
            pl.BlockSpec((NC, BM, D), lambda i: (0, i, 0)),
            pl.BlockSpec((1, 1, BM), lambda i: (i, 0, 0)),
            pl.BlockSpec((1, D), lambda i: (0, 0)),
            pl.BlockSpec((D, D), lambda i: (0, 0)),
        ],
        out_specs=pl.BlockSpec((BM, D), lambda i: (i, 0)),
        out_shape=jax.ShapeDtypeStruct((NP, D), jnp.float32),
    )(agg, dinv3d, b1r, W2)


def _tc_final(agg2, dinv3d, b2r, batch3d, Wl, blr):
    def body(a_ref, dinv_ref, b_ref, bat_ref, wl_ref, bl_ref, out_ref,
             sums, cnts):
        i = pl.program_id(0)

        @pl.when(i == 0)
        def _():
            sums[...] = jnp.zeros_like(sums)
            cnts[...] = jnp.zeros_like(cnts)

        dinv = dinv_ref[0, 0, :]
        h2 = (a_ref[0] + a_ref[1]) * dinv[:, None] + b_ref[...]
        bat = bat_ref[0, 0, :]
        gids = lax.broadcasted_iota(jnp.int32, (1, NG), 1)
        mask = (bat[:, None] == gids).astype(jnp.float32)
        sums[...] += lax.dot_general(mask, h2, (((0,), (0,)), ((), ())),
                                     precision=_PREC,
                                     preferred_element_type=jnp.float32)
        cnts[...] += lax.dot_general(mask, jnp.ones_like(h2),
                                     (((0,), (0,)), ((), ())),
                                     precision=_PREC,
                                     preferred_element_type=jnp.float32)

        @pl.when(i == NB - 1)
        def _():
            pooled = sums[...] / jnp.maximum(cnts[...], 1.0)
            out_ref[...] = jnp.dot(pooled, wl_ref[...], precision=_PREC,
                                   preferred_element_type=jnp.float32) + bl_ref[...]

    return pl.pallas_call(
        body,
        grid=(NB,),
        in_specs=[
            pl.BlockSpec((NC, BM, D), lambda i: (0, i, 0)),
            pl.BlockSpec((1, 1, BM), lambda i: (i, 0, 0)),
            pl.BlockSpec((1, D), lambda i: (0, 0)),
            pl.BlockSpec((1, 1, BM), lambda i: (i, 0, 0)),
            pl.BlockSpec((D, D), lambda i: (0, 0)),
            pl.BlockSpec((1, D), lambda i: (0, 0)),
        ],
        out_specs=pl.BlockSpec((NG, D), lambda i: (0, 0)),
        out_shape=jax.ShapeDtypeStruct((NG, D), jnp.float32),
        scratch_shapes=[
            pltpu.VMEM((NG, D), jnp.float32),
            pltpu.VMEM((NG, D), jnp.float32),
        ],
    )(agg2, dinv3d, b2r, batch3d, Wl, blr)


@jax.jit
def _run(x, edge_index, batch, W1, b1, W2, b2, Wl, bl):
    n = x.shape[0]
    e = edge_index.shape[1]

    pad_idx = jnp.full((EP - e,), n, dtype=jnp.int32)
    src = jnp.concatenate([edge_index[0].astype(jnp.int32), pad_idx])
    dst = jnp.concatenate([edge_index[1].astype(jnp.int32), pad_idx])
    x_pad = jnp.concatenate(
        [x, jnp.zeros((NP - n, x.shape[1]), dtype=x.dtype)])
    batch_pad = jnp.concatenate(
        [batch.astype(jnp.int32), jnp.full((NP - n,), NG, dtype=jnp.int32)])
    batch3d = batch_pad.reshape(NB, 1, BM)

    zeros128 = jnp.zeros((NP, D), dtype=jnp.float32)
    b1r = b1.reshape(1, D)
    b2r = b2.reshape(1, D)
    blr = bl.reshape(1, D)

    cnt = _sc_count(dst, zeros128)
    g1, dinv3d = _tc_g1(x_pad, W1, cnt)
    agg1 = _sc_agg(src, dst, g1, zeros128)
    g2 = _tc_combine(agg1, dinv3d, b1r, W2)
    agg2 = _sc_agg(src, dst, g2, zeros128)
    return _tc_final(agg2, dinv3d, b2r, batch3d, Wl, blr)


def kernel(x, edge_index, edge_attr, batch, W1, b1, W2, b2, Wl, bl):
    del edge_attr
    return _run(x, edge_index, batch, W1, b1, W2, b2, Wl, bl)


# speedup vs baseline: 25.7596x; 2.8690x over previous
import functools

import jax
import jax.numpy as jnp
from jax import lax
from jax.experimental import pallas as pl
from jax.experimental.pallas import tpu as pltpu
from jax.experimental.pallas import tpu_sc as plsc

NC = 2
NS = 16
NW = NC * NS
CHUNK = 128
NP = 10240
EP = 327680
EPW = EP // NW
NPT = NP // NS
BM = 2048
NB = NP // BM
NG = 16
D = 128

_PREC = lax.Precision.HIGHEST


def _mesh():
    return plsc.VectorSubcoreMesh(core_axis_name="c", subcore_axis_name="s")


def _sc_count(dst, zeros128):
    @functools.partial(
        pl.kernel,
        out_type=jax.ShapeDtypeStruct((NC, NP, D), jnp.float32),
        mesh=_mesh(),
        scratch_types=[
            pltpu.VMEM((CHUNK,), jnp.int32),
            pltpu.VMEM((CHUNK, D), jnp.float32),
            pltpu.VMEM_SHARED((NP, D), jnp.float32),
        ],
    )
    def body(dst_hbm, z_hbm, out_hbm, idx_v, ones_v, acc_sh):
        c = lax.axis_index("c")
        s = lax.axis_index("s")
        wid = s * NC + c

        def fill(r, carry):
            def fill16(j, carry2):
                ones_v[r, pl.ds(j * 16, 16)] = jnp.full((16,), 1.0,
                                                        jnp.float32)
                return carry2
            return lax.fori_loop(0, D // 16, fill16, carry)

        lax.fori_loop(0, CHUNK, fill, 0)
        pltpu.sync_copy(z_hbm.at[pl.ds(s * NPT, NPT)],
                        acc_sh.at[pl.ds(s * NPT, NPT)])
        plsc.subcore_barrier()
        base = wid * EPW

        def step(k, carry):
            pltpu.sync_copy(dst_hbm.at[pl.ds(base + k * CHUNK, CHUNK)], idx_v)
            pltpu.sync_copy(ones_v, acc_sh.at[idx_v], add=True)
            return carry

        lax.fori_loop(0, EPW // CHUNK, step, 0)
        plsc.subcore_barrier()
        pltpu.sync_copy(acc_sh.at[pl.ds(s * NPT, NPT)],
                        out_hbm.at[c, pl.ds(s * NPT, NPT)])

    return body(dst, zeros128)


def _sc_agg(src, dst, g, zeros128):
    nk = EPW // CHUNK
    GK = 8
    ngrp = nk // GK

    @functools.partial(
        pl.kernel,
        out_type=jax.ShapeDtypeStruct((NC, NP, D), jnp.float32),
        mesh=_mesh(),
        scratch_types=[
            pltpu.VMEM((2, GK, CHUNK), jnp.int32),
            pltpu.VMEM((2, GK, CHUNK), jnp.int32),
            pltpu.VMEM((CHUNK, D), jnp.float32),
            pltpu.VMEM((CHUNK, D), jnp.float32),
            pltpu.SemaphoreType.DMA,
            pltpu.SemaphoreType.DMA,
            pltpu.SemaphoreType.DMA,
            pltpu.SemaphoreType.DMA,
            pltpu.SemaphoreType.DMA,
            pltpu.VMEM_SHARED((NP, D), jnp.float32),
        ],
    )
    def body(src_hbm, dst_hbm, g_hbm, z_hbm, out_hbm,
             sidx_v, didx_v, rows_a, rows_b, gsa, gsb, ssa, ssb, isem,
             acc_sh):
        c = lax.axis_index("c")
        s = lax.axis_index("s")
        wid = s * NC + c
        r0 = s * NPT
        base = wid * nk
        rows = (rows_a, rows_b)
        gsem = (gsa, gsb)
        ssem = (ssa, ssb)

        def load_group(gi):
            gb = gi % 2
            pltpu.async_copy(src_hbm.at[pl.ds(base + gi * GK, GK)],
                             sidx_v.at[gb], isem)
            pltpu.async_copy(dst_hbm.at[pl.ds(base + gi * GK, GK)],
                             didx_v.at[gb], isem)

        def wait_group(gi):
            gb = gi % 2
            pltpu.make_async_copy(src_hbm.at[pl.ds(base + gi * GK, GK)],
                                  sidx_v.at[gb], isem).wait()
            pltpu.make_async_copy(dst_hbm.at[pl.ds(base + gi * GK, GK)],
                                  didx_v.at[gb], isem).wait()

        load_group(0)

        @pl.when(c == 0)
        def _():
            pltpu.sync_copy(g_hbm.at[pl.ds(r0, NPT)], acc_sh.at[pl.ds(r0, NPT)])

        @pl.when(c != 0)
        def _():
            pltpu.sync_copy(z_hbm.at[pl.ds(r0, NPT)], acc_sh.at[pl.ds(r0, NPT)])

        wait_group(0)
        plsc.subcore_barrier()

        pltpu.async_copy(g_hbm.at[sidx_v.at[0, 0]], rows_a, gsa)
        for k in range(nk):
            b = k & 1
            gi, r = divmod(k, GK)
            if k >= 1 and k + 1 < nk:
                pltpu.make_async_copy(
                    rows[1 - b],
                    acc_sh.at[didx_v.at[(k - 1) // GK % 2, (k - 1) % GK]],
                    ssem[1 - b]).wait()
            if r == 1 and gi + 1 < ngrp:
                load_group(gi + 1)
            if k + 1 < nk:
                ngi, nr = divmod(k + 1, GK)
                if nr == 0:
                    wait_group(ngi)
                pltpu.async_copy(g_hbm.at[sidx_v.at[ngi % 2, nr]],
                                 rows[1 - b], gsem[1 - b])
            pltpu.make_async_copy(g_hbm.at[sidx_v.at[gi % 2, r]],
                                  rows[b], gsem[b]).wait()
            pltpu.async_copy(rows[b], acc_sh.at[didx_v.at[gi % 2, r]],
                             ssem[b], add=True)
        for k in (nk - 2, nk - 1):
            b = k & 1
            gi, r = divmod(k, GK)
            pltpu.make_async_copy(rows[b], acc_sh.at[didx_v.at[gi % 2, r]],
                                  ssem[b]).wait()
        plsc.subcore_barrier()
        pltpu.sync_copy(acc_sh.at[pl.ds(r0, NPT)],
                        out_hbm.at[c, pl.ds(r0, NPT)])

    return body(src.reshape(EP // CHUNK, CHUNK),
                dst.reshape(EP // CHUNK, CHUNK), g, zeros128)


def _tc_g1(x_pad, W1, cnt):
    def body(x_ref, w_ref, cnt_ref, g_ref, dinv_ref):
        cntv = cnt_ref[0, :, 0] + cnt_ref[1, :, 0] + 1.0
        dinv = lax.rsqrt(cntv)
        h = jnp.dot(x_ref[...], w_ref[...], precision=_PREC,
                    preferred_element_type=jnp.float32)
        g_ref[...] = h * dinv[:, None]
        dinv_ref[0, 0, :] = dinv

    return pl.pallas_call(
        body,
        grid=(NB,),
        in_specs=[
            pl.BlockSpec((BM, D), lambda i: (i, 0)),
            pl.BlockSpec((D, D), lambda i: (0, 0)),
            pl.BlockSpec((NC, BM, D), lambda i: (0, i, 0)),
        ],
        out_specs=[
            pl.BlockSpec((BM, D), lambda i: (i, 0)),
            pl.BlockSpec((1, 1, BM), lambda i: (i, 0, 0)),
        ],
        out_shape=[
            jax.ShapeDtypeStruct((NP, D), jnp.float32),
            jax.ShapeDtypeStruct((NB, 1, BM), jnp.float32),
        ],
    )(x_pad, W1, cnt)


def _tc_combine(agg, dinv3d, b1r, W2):
    def body(a_ref, dinv_ref, b_ref, w_ref, g_ref):
        dinv = dinv_ref[0, 0, :]
        p = a_ref[0] + a_ref[1]
        h = jnp.maximum(p * dinv[:, None] + b_ref[...], 0.0)
        g_ref[...] = jnp.dot(h, w_ref[...], precision=_PREC,
                             preferred_element_type=jnp.float32) * dinv[:, None]

    return pl.pallas_call(
        body,
        grid=(NB,),
        in_specs=[
            pl.BlockSpec((NC, BM, D), lambda i: (0, i, 0)),
            pl.BlockSpec((1, 1, BM), lambda i: (i, 0, 0)),
            pl.BlockSpec((1, D), lambda i: (0, 0)),
            pl.BlockSpec((D, D), lambda i: (0, 0)),
        ],
        out_specs=pl.BlockSpec((BM, D), lambda i: (i, 0)),
        out_shape=jax.ShapeDtypeStruct((NP, D), jnp.float32),
    )(agg, dinv3d, b1r, W2)


def _tc_final(agg2, dinv3d, b2r, batch3d, Wl, blr):
    def body(a_ref, dinv_ref, b_ref, bat_ref, wl_ref, bl_ref, out_ref,
             sums, cnts):
        i = pl.program_id(0)

        @pl.when(i == 0)
        def _():
            sums[...] = jnp.zeros_like(sums)
            cnts[...] = jnp.zeros_like(cnts)

        dinv = dinv_ref[0, 0, :]
        h2 = (a_ref[0] + a_ref[1]) * dinv[:, None] + b_ref[...]
        bat = bat_ref[0, 0, :]
        gids = lax.broadcasted_iota(jnp.int32, (1, NG), 1)
        mask = (bat[:, None] == gids).astype(jnp.float32)
        sums[...] += lax.dot_general(mask, h2, (((0,), (0,)), ((), ())),
                                     precision=_PREC,
                                     preferred_element_type=jnp.float32)
        cnts[...] += lax.dot_general(mask, jnp.ones_like(h2),
                                     (((0,), (0,)), ((), ())),
                                     precision=_PREC,
                                     preferred_element_type=jnp.float32)

        @pl.when(i == NB - 1)
        def _():
            pooled = sums[...] / jnp.maximum(cnts[...], 1.0)
            out_ref[...] = jnp.dot(pooled, wl_ref[...], precision=_PREC,
                                   preferred_element_type=jnp.float32) + bl_ref[...]

    return pl.pallas_call(
        body,
        grid=(NB,),
        in_specs=[
            pl.BlockSpec((NC, BM, D), lambda i: (0, i, 0)),
            pl.BlockSpec((1, 1, BM), lambda i: (i, 0, 0)),
            pl.BlockSpec((1, D), lambda i: (0, 0)),
            pl.BlockSpec((1, 1, BM), lambda i: (i, 0, 0)),
            pl.BlockSpec((D, D), lambda i: (0, 0)),
            pl.BlockSpec((1, D), lambda i: (0, 0)),
        ],
        out_specs=pl.BlockSpec((NG, D), lambda i: (0, 0)),
        out_shape=jax.ShapeDtypeStruct((NG, D), jnp.float32),
        scratch_shapes=[
            pltpu.VMEM((NG, D), jnp.float32),
            pltpu.VMEM((NG, D), jnp.float32),
        ],
    )(agg2, dinv3d, b2r, batch3d, Wl, blr)


@jax.jit
def _run(x, edge_index, batch, W1, b1, W2, b2, Wl, bl):
    n = x.shape[0]
    e = edge_index.shape[1]

    pad_idx = n + jnp.arange(EP - e, dtype=jnp.int32) % (NP - n)
    src = jnp.concatenate([edge_index[0].astype(jnp.int32), pad_idx])
    dst = jnp.concatenate([edge_index[1].astype(jnp.int32), pad_idx])
    x_pad = jnp.concatenate(
        [x, jnp.zeros((NP - n, x.shape[1]), dtype=x.dtype)])
    batch_pad = jnp.concatenate(
        [batch.astype(jnp.int32), jnp.full((NP - n,), NG, dtype=jnp.int32)])
    batch3d = batch_pad.reshape(NB, 1, BM)

    zeros128 = jnp.zeros((NP, D), dtype=jnp.float32)
    b1r = b1.reshape(1, D)
    b2r = b2.reshape(1, D)
    blr = bl.reshape(1, D)

    cnt = _sc_count(dst, zeros128)
    g1, dinv3d = _tc_g1(x_pad, W1, cnt)
    agg1 = _sc_agg(src, dst, g1, zeros128)
    g2 = _tc_combine(agg1, dinv3d, b1r, W2)
    agg2 = _sc_agg(src, dst, g2, zeros128)
    return _tc_final(agg2, dinv3d, b2r, batch3d, Wl, blr)


def kernel(x, edge_index, edge_attr, batch, W1, b1, W2, b2, Wl, bl):
    del edge_attr
    return _run(x, edge_index, batch, W1, b1, W2, b2, Wl, bl)


# speedup vs baseline: 26.1789x; 1.0163x over previous
import functools

import jax
import jax.numpy as jnp
from jax import lax
from jax.experimental import pallas as pl
from jax.experimental.pallas import tpu as pltpu
from jax.experimental.pallas import tpu_sc as plsc

NC = 2
NS = 16
NW = NC * NS
CHUNK = 128
NP = 10240
EP = 327680
EPW = EP // NW
NPT = NP // NS
BM = 2048
NB = NP // BM
NG = 16
D = 128

_PREC = lax.Precision.HIGHEST


def _mesh():
    return plsc.VectorSubcoreMesh(core_axis_name="c", subcore_axis_name="s")


def _sc_count(dst, zeros128):
    @functools.partial(
        pl.kernel,
        out_type=jax.ShapeDtypeStruct((NC, NP, D), jnp.float32),
        mesh=_mesh(),
        scratch_types=[
            pltpu.VMEM((CHUNK,), jnp.int32),
            pltpu.VMEM((CHUNK, D), jnp.float32),
            pltpu.VMEM_SHARED((NP, D), jnp.float32),
        ],
    )
    def body(dst_hbm, z_hbm, out_hbm, idx_v, ones_v, acc_sh):
        c = lax.axis_index("c")
        s = lax.axis_index("s")
        wid = s * NC + c

        def fill(r, carry):
            def fill16(j, carry2):
                ones_v[r, pl.ds(j * 16, 16)] = jnp.full((16,), 1.0,
                                                        jnp.float32)
                return carry2
            return lax.fori_loop(0, D // 16, fill16, carry)

        lax.fori_loop(0, CHUNK, fill, 0)
        pltpu.sync_copy(z_hbm.at[pl.ds(s * NPT, NPT)],
                        acc_sh.at[pl.ds(s * NPT, NPT)])
        plsc.subcore_barrier()
        base = wid * EPW

        def step(k, carry):
            pltpu.sync_copy(dst_hbm.at[pl.ds(base + k * CHUNK, CHUNK)], idx_v)
            pltpu.sync_copy(ones_v, acc_sh.at[idx_v], add=True)
            return carry

        lax.fori_loop(0, EPW // CHUNK, step, 0)
        plsc.subcore_barrier()
        pltpu.sync_copy(acc_sh.at[pl.ds(s * NPT, NPT)],
                        out_hbm.at[c, pl.ds(s * NPT, NPT)])

    return body(dst, zeros128)


def _sc_agg(src, dst, g):
    nk = EPW // CHUNK
    GK = 8
    ngrp = nk // GK

    @functools.partial(
        pl.kernel,
        out_type=jax.ShapeDtypeStruct((NC, NP, D), jnp.float32),
        mesh=_mesh(),
        scratch_types=[
            pltpu.VMEM((2, GK, CHUNK), jnp.int32),
            pltpu.VMEM((2, GK, CHUNK), jnp.int32),
            pltpu.VMEM((CHUNK, D), jnp.float32),
            pltpu.VMEM((CHUNK, D), jnp.float32),
            pltpu.SemaphoreType.DMA,
            pltpu.SemaphoreType.DMA,
            pltpu.SemaphoreType.DMA,
            pltpu.SemaphoreType.DMA,
            pltpu.SemaphoreType.DMA,
            pltpu.VMEM_SHARED((NP, D), jnp.float32),
        ],
    )
    def body(src_hbm, dst_hbm, g_hbm, out_hbm,
             sidx_v, didx_v, rows_a, rows_b, gsa, gsb, ssa, ssb, isem,
             acc_sh):
        c = lax.axis_index("c")
        s = lax.axis_index("s")
        wid = s * NC + c
        r0 = s * NPT
        base = wid * nk
        rows = (rows_a, rows_b)
        gsem = (gsa, gsb)
        ssem = (ssa, ssb)

        def load_group(gi):
            gb = gi % 2
            pltpu.async_copy(src_hbm.at[pl.ds(base + gi * GK, GK)],
                             sidx_v.at[gb], isem)
            pltpu.async_copy(dst_hbm.at[pl.ds(base + gi * GK, GK)],
                             didx_v.at[gb], isem)

        def wait_group(gi):
            gb = gi % 2
            pltpu.make_async_copy(src_hbm.at[pl.ds(base + gi * GK, GK)],
                                  sidx_v.at[gb], isem).wait()
            pltpu.make_async_copy(dst_hbm.at[pl.ds(base + gi * GK, GK)],
                                  didx_v.at[gb], isem).wait()

        load_group(0)
        wait_group(0)
        pltpu.async_copy(g_hbm.at[sidx_v.at[0, 0]], rows_a, gsa)

        @pl.when(c == 0)
        def _():
            pltpu.sync_copy(g_hbm.at[pl.ds(r0, NPT)], acc_sh.at[pl.ds(r0, NPT)])

        @pl.when(c != 0)
        def _():
            def zfill(r, carry):
                def z16(j, carry2):
                    rows_b[r, pl.ds(j * 16, 16)] = jnp.zeros((16,), jnp.float32)
                    return carry2
                return lax.fori_loop(0, D // 16, z16, carry)

            lax.fori_loop(0, CHUNK, zfill, 0)
            for t in range(NPT // CHUNK):
                pltpu.sync_copy(rows_b,
                                acc_sh.at[pl.ds(r0 + t * CHUNK, CHUNK)])

        plsc.subcore_barrier()

        for k in range(nk):
            b = k & 1
            gi, r = divmod(k, GK)
            if k >= 1 and k + 1 < nk:
                pltpu.make_async_copy(
                    rows[1 - b],
                    acc_sh.at[didx_v.at[(k - 1) // GK % 2, (k - 1) % GK]],
                    ssem[1 - b]).wait()
            if r == 1 and gi + 1 < ngrp:
                load_group(gi + 1)
            if k + 1 < nk:
                ngi, nr = divmod(k + 1, GK)
                if nr == 0:
                    wait_group(ngi)
                pltpu.async_copy(g_hbm.at[sidx_v.at[ngi % 2, nr]],
                                 rows[1 - b], gsem[1 - b])
            pltpu.make_async_copy(g_hbm.at[sidx_v.at[gi % 2, r]],
                                  rows[b], gsem[b]).wait()
            pltpu.async_copy(rows[b], acc_sh.at[didx_v.at[gi % 2, r]],
                             ssem[b], add=True)
        for k in (nk - 2, nk - 1):
            b = k & 1
            gi, r = divmod(k, GK)
            pltpu.make_async_copy(rows[b], acc_sh.at[didx_v.at[gi % 2, r]],
                                  ssem[b]).wait()
        plsc.subcore_barrier()
        pltpu.sync_copy(acc_sh.at[pl.ds(r0, NPT)],
                        out_hbm.at[c, pl.ds(r0, NPT)])

    return body(src.reshape(EP // CHUNK, CHUNK),
                dst.reshape(EP // CHUNK, CHUNK), g)


def _tc_mm1(x_pad, W1):
    def body(x_ref, w_ref, h_ref):
        h_ref[...] = jnp.dot(x_ref[...], w_ref[...], precision=_PREC,
                             preferred_element_type=jnp.float32)

    return pl.pallas_call(
        body,
        grid=(NB,),
        in_specs=[
            pl.BlockSpec((BM, D), lambda i: (i, 0)),
            pl.BlockSpec((D, D), lambda i: (0, 0)),
        ],
        out_specs=pl.BlockSpec((BM, D), lambda i: (i, 0)),
        out_shape=jax.ShapeDtypeStruct((NP, D), jnp.float32),
    )(x_pad, W1)


def _tc_scale(h1, cnt):
    def body(h_ref, cnt_ref, g_ref, dinv_ref):
        cntv = cnt_ref[0, :, 0] + cnt_ref[1, :, 0] + 1.0
        dinv = lax.rsqrt(cntv)
        g_ref[...] = h_ref[...] * dinv[:, None]
        dinv_ref[0, 0, :] = dinv

    return pl.pallas_call(
        body,
        grid=(NB,),
        in_specs=[
            pl.BlockSpec((BM, D), lambda i: (i, 0)),
            pl.BlockSpec((NC, BM, D), lambda i: (0, i, 0)),
        ],
        out_specs=[
            pl.BlockSpec((BM, D), lambda i: (i, 0)),
            pl.BlockSpec((1, 1, BM), lambda i: (i, 0, 0)),
        ],
        out_shape=[
            jax.ShapeDtypeStruct((NP, D), jnp.float32),
            jax.ShapeDtypeStruct((NB, 1, BM), jnp.float32),
        ],
    )(h1, cnt)


def _tc_combine(agg, dinv3d, b1r, W2):
    def body(a_ref, dinv_ref, b_ref, w_ref, g_ref):
        dinv = dinv_ref[0, 0, :]
        p = a_ref[0] + a_ref[1]
        h = jnp.maximum(p * dinv[:, None] + b_ref[...], 0.0)
        g_ref[...] = jnp.dot(h, w_ref[...], precision=_PREC,
                             preferred_element_type=jnp.float32) * dinv[:, None]

    return pl.pallas_call(
        body,
        grid=(NB,),
        in_specs=[
            pl.BlockSpec((NC, BM, D), lambda i: (0, i, 0)),
            pl.BlockSpec((1, 1, BM), lambda i: (i, 0, 0)),
            pl.BlockSpec((1, D), lambda i: (0, 0)),
            pl.BlockSpec((D, D), lambda i: (0, 0)),
        ],
        out_specs=pl.BlockSpec((BM, D), lambda i: (i, 0)),
        out_shape=jax.ShapeDtypeStruct((NP, D), jnp.float32),
    )(agg, dinv3d, b1r, W2)


def _tc_final(agg2, dinv3d, b2r, batch3d, Wl, blr):
    def body(a_ref, dinv_ref, b_ref, bat_ref, wl_ref, bl_ref, out_ref,
             sums, cnts):
        i = pl.program_id(0)

        @pl.when(i == 0)
        def _():
            sums[...] = jnp.zeros_like(sums)
            cnts[...] = jnp.zeros_like(cnts)

        dinv = dinv_ref[0, 0, :]
        h2 = (a_ref[0] + a_ref[1]) * dinv[:, None] + b_ref[...]
        bat = bat_ref[0, 0, :]
        gids = lax.broadcasted_iota(jnp.int32, (1, NG), 1)
        mask = (bat[:, None] == gids).astype(jnp.float32)
        sums[...] += lax.dot_general(mask, h2, (((0,), (0,)), ((), ())),
                                     precision=_PREC,
                                     preferred_element_type=jnp.float32)
        cnts[...] += lax.dot_general(mask, jnp.ones_like(h2),
                                     (((0,), (0,)), ((), ())),
                                     precision=_PREC,
                                     preferred_element_type=jnp.float32)

        @pl.when(i == NB - 1)
        def _():
            pooled = sums[...] / jnp.maximum(cnts[...], 1.0)
            out_ref[...] = jnp.dot(pooled, wl_ref[...], precision=_PREC,
                                   preferred_element_type=jnp.float32) + bl_ref[...]

    return pl.pallas_call(
        body,
        grid=(NB,),
        in_specs=[
            pl.BlockSpec((NC, BM, D), lambda i: (0, i, 0)),
            pl.BlockSpec((1, 1, BM), lambda i: (i, 0, 0)),
            pl.BlockSpec((1, D), lambda i: (0, 0)),
            pl.BlockSpec((1, 1, BM), lambda i: (i, 0, 0)),
            pl.BlockSpec((D, D), lambda i: (0, 0)),
            pl.BlockSpec((1, D), lambda i: (0, 0)),
        ],
        out_specs=pl.BlockSpec((NG, D), lambda i: (0, 0)),
        out_shape=jax.ShapeDtypeStruct((NG, D), jnp.float32),
        scratch_shapes=[
            pltpu.VMEM((NG, D), jnp.float32),
            pltpu.VMEM((NG, D), jnp.float32),
        ],
    )(agg2, dinv3d, b2r, batch3d, Wl, blr)


@jax.jit
def _run(x, edge_index, batch, W1, b1, W2, b2, Wl, bl):
    n = x.shape[0]
    e = edge_index.shape[1]

    pad_idx = n + jnp.arange(EP - e, dtype=jnp.int32) % (NP - n)
    src = jnp.concatenate([edge_index[0].astype(jnp.int32), pad_idx])
    dst = jnp.concatenate([edge_index[1].astype(jnp.int32), pad_idx])
    x_pad = jnp.concatenate(
        [x, jnp.zeros((NP - n, x.shape[1]), dtype=x.dtype)])
    batch_pad = jnp.concatenate(
        [batch.astype(jnp.int32), jnp.full((NP - n,), NG, dtype=jnp.int32)])
    batch3d = batch_pad.reshape(NB, 1, BM)

    zeros128 = jnp.zeros((NP, D), dtype=jnp.float32)
    b1r = b1.reshape(1, D)
    b2r = b2.reshape(1, D)
    blr = bl.reshape(1, D)

    cnt = _sc_count(dst, zeros128)
    h1 = _tc_mm1(x_pad, W1)
    g1, dinv3d = _tc_scale(h1, cnt)
    agg1 = _sc_agg(src, dst, g1)
    g2 = _tc_combine(agg1, dinv3d, b1r, W2)
    agg2 = _sc_agg(src, dst, g2)
    return _tc_final(agg2, dinv3d, b2r, batch3d, Wl, blr)


def kernel(x, edge_index, edge_attr, batch, W1, b1, W2, b2, Wl, bl):
    del edge_attr
    return _run(x, edge_index, batch, W1, b1, W2, b2, Wl, bl)
